# baseline (speedup 1.0000x reference)
import functools

import jax
import jax.numpy as jnp
from jax import lax
from jax.experimental import pallas as pl
from jax.experimental.pallas import tpu as pltpu
from jax.experimental.pallas import tpu_sc as plsc

F = 26
B = 16384
V = 100000
D = 64
MLP_IN = F * D

NC = 2
NS = 16
NW = NC * NS
RPW = MLP_IN // NW
OCH = 4096
NOC = B // OCH
GRP = 8


def _sc_gather_t(tbl_t, idx):
  mesh = plsc.VectorSubcoreMesh(
      core_axis_name="c", subcore_axis_name="s", num_cores=NC, num_subcores=NS)

  @functools.partial(
      pl.kernel,
      mesh=mesh,
      out_type=jax.ShapeDtypeStruct((MLP_IN, B), jnp.float32),
      scratch_types=[
          pltpu.VMEM((1, V), jnp.float32),
          pltpu.VMEM((1, B), jnp.int32),
          pltpu.VMEM((2, OCH), jnp.float32),
          pltpu.SemaphoreType.DMA,
          pltpu.SemaphoreType.DMA,
          pltpu.SemaphoreType.DMA((2,)),
      ],
      compiler_params=pltpu.CompilerParams(
          use_tc_tiling_on_sc=True, needs_layout_passes=False),
  )
  def k(tbl_hbm, idx_hbm, out_hbm, row_v, idx_v, out_v, rsem, isem, osems):
    wid = lax.axis_index("s") * NC + lax.axis_index("c")
    row0 = wid * RPW

    def load_idx(f):
      pltpu.make_async_copy(idx_hbm.at[pl.ds(f, 1)], idx_v, isem).start()
      pltpu.make_async_copy(idx_hbm.at[pl.ds(f, 1)], idx_v, isem).wait()

    load_idx(row0 // D)

    def row_step(r, f_loaded):
      fd = row0 + r
      f = fd // D
      pltpu.make_async_copy(tbl_hbm.at[pl.ds(fd, 1)], row_v, rsem).start()

      @pl.when(f != f_loaded)
      def _():
        load_idx(f)

      pltpu.make_async_copy(tbl_hbm.at[pl.ds(fd, 1)], row_v, rsem).wait()

      for c in range(NOC):
        slot = c % 2
        @pl.when((fd > row0) | (c >= 2))
        def _():
          pltpu.make_async_copy(
              out_v.at[pl.ds(slot, 1)],
              out_hbm.at[pl.ds(0, 1), pl.ds(0, OCH)], osems.at[slot]).wait()

        def grp_body(j, carry):
          ids_l = [idx_v[0, pl.ds(c * OCH + (j * GRP + u) * 16, 16)]
                   for u in range(GRP)]
          vals_l = [plsc.load_gather(row_v.at[0], [ids]) for ids in ids_l]
          for u in range(GRP):
            out_v[slot, pl.ds((j * GRP + u) * 16, 16)] = vals_l[u]
          return carry

        lax.fori_loop(0, OCH // (16 * GRP), grp_body, 0)
        pltpu.make_async_copy(
            out_v.at[pl.ds(slot, 1)],
            out_hbm.at[pl.ds(fd, 1), pl.ds(c * OCH, OCH)],
            osems.at[slot]).start()
      return f

    lax.fori_loop(0, RPW, row_step, row0 // D)
    for slot in range(2):
      pltpu.make_async_copy(
          out_v.at[pl.ds(slot, 1)],
          out_hbm.at[pl.ds(0, 1), pl.ds(0, OCH)], osems.at[slot]).wait()

  return k(tbl_t, idx)


def _tc_dense_t(embT, W1, b1c, W2, b2c, W3, b3c, W4, b4c, S):
  bB = 512
  grid = (B // bB,)
  dn0 = (((0,), (0,)), ((), ()))

  def body(x_ref, w1_ref, b1_ref, w2_ref, b2_ref, w3_ref, b3_ref, w4_ref,
           b4_ref, s_ref, o_ref):
    x = x_ref[...]
    sum_e = lax.dot_general(s_ref[...], x, dn0,
                            preferred_element_type=jnp.float32)
    sq_e = lax.dot_general(s_ref[...], x * x, dn0,
                           preferred_element_type=jnp.float32)
    fm = 0.5 * jnp.sum(sum_e * sum_e - sq_e, axis=0, keepdims=True)
    h = jnp.maximum(
        lax.dot_general(w1_ref[...], x, dn0,
                        preferred_element_type=jnp.float32) + b1_ref[...], 0.0)
    h = jnp.maximum(
        lax.dot_general(w2_ref[...], h, dn0,
                        preferred_element_type=jnp.float32) + b2_ref[...], 0.0)
    h = jnp.maximum(
        lax.dot_general(w3_ref[...], h, dn0,
                        preferred_element_type=jnp.float32) + b3_ref[...], 0.0)
    deep = lax.dot_general(w4_ref[...], h, dn0,
                           preferred_element_type=jnp.float32)
    o_ref[...] = fm + deep + b4_ref[...]

  full = lambda shape: pl.BlockSpec(shape, lambda i: (0,) * len(shape))
  return pl.pallas_call(
      body,
      grid=grid,
      in_specs=[
          pl.BlockSpec((MLP_IN, bB), lambda i: (0, i)),
          full((MLP_IN, 256)),
          full((256, 1)),
          full((256, 128)),
          full((128, 1)),
          full((128, 64)),
          full((64, 1)),
          full((64, 1)),
          full((1, 1)),
          full((MLP_IN, D)),
      ],
      out_specs=pl.BlockSpec((1, bB), lambda i: (0, i)),
      out_shape=jax.ShapeDtypeStruct((1, B), jnp.float32),
  )(embT, W1, b1c, W2, b2c, W3, b3c, W4, b4c, S)


@jax.jit
def kernel(sparse_indices_list, tables, W1, b1, W2, b2, W3, b3, W4, b4):
  tbl_t = tables.transpose(0, 2, 1).reshape(MLP_IN, V)
  idx = sparse_indices_list.astype(jnp.int32)

  embT = _sc_gather_t(tbl_t, idx)

  S = jnp.tile(jnp.eye(D, dtype=jnp.float32), (F, 1))
  logits = _tc_dense_t(embT, W1, b1.reshape(256, 1), W2, b2.reshape(128, 1),
                       W3, b3.reshape(64, 1), W4, b4.reshape(1, 1), S)
  return logits.reshape(B)


# speedup vs baseline: 1.0305x; 1.0305x over previous
import functools

import jax
import jax.numpy as jnp
from jax import lax
from jax.experimental import pallas as pl
from jax.experimental.pallas import tpu as pltpu
from jax.experimental.pallas import tpu_sc as plsc

F = 26
B = 16384
V = 100000
D = 64
MLP_IN = F * D

NC = 2
NS = 16
NW = NC * NS
RPW = MLP_IN // NW
OCH = 4096
NOC = B // OCH
GRP = 8


def _sc_gather_t(tbl_t, idx):
  mesh = plsc.VectorSubcoreMesh(
      core_axis_name="c", subcore_axis_name="s", num_cores=NC, num_subcores=NS)

  @functools.partial(
      pl.kernel,
      mesh=mesh,
      out_type=jax.ShapeDtypeStruct((MLP_IN, B), jnp.float32),
      scratch_types=[
          pltpu.VMEM((1, V), jnp.float32),
          pltpu.VMEM((1, B), jnp.int32),
          pltpu.VMEM((2, OCH), jnp.float32),
          pltpu.SemaphoreType.DMA,
          pltpu.SemaphoreType.DMA,
          pltpu.SemaphoreType.DMA((2,)),
      ],
      compiler_params=pltpu.CompilerParams(
          use_tc_tiling_on_sc=True, needs_layout_passes=False),
  )
  def k(tbl_hbm, idx_hbm, out_hbm, row_v, idx_v, out_v, rsem, isem, osems):
    wid = lax.axis_index("s") * NC + lax.axis_index("c")
    row0 = wid * RPW

    def load_idx(f):
      pltpu.make_async_copy(idx_hbm.at[pl.ds(f, 1)], idx_v, isem).start()
      pltpu.make_async_copy(idx_hbm.at[pl.ds(f, 1)], idx_v, isem).wait()

    load_idx(row0 // D)

    def row_step(r, f_loaded):
      fd = row0 + r
      f = fd // D
      pltpu.make_async_copy(tbl_hbm.at[pl.ds(fd, 1)], row_v, rsem).start()

      @pl.when(f != f_loaded)
      def _():
        load_idx(f)

      pltpu.make_async_copy(tbl_hbm.at[pl.ds(fd, 1)], row_v, rsem).wait()

      for c in range(NOC):
        slot = c % 2
        @pl.when((fd > row0) | (c >= 2))
        def _():
          pltpu.make_async_copy(
              out_v.at[pl.ds(slot, 1)],
              out_hbm.at[pl.ds(0, 1), pl.ds(0, OCH)], osems.at[slot]).wait()

        def grp_body(j, carry):
          ids_l = [idx_v[0, pl.ds(c * OCH + (j * GRP + u) * 16, 16)]
                   for u in range(GRP)]
          vals_l = [plsc.load_gather(row_v.at[0], [ids]) for ids in ids_l]
          for u in range(GRP):
            out_v[slot, pl.ds((j * GRP + u) * 16, 16)] = vals_l[u]
          return carry

        lax.fori_loop(0, OCH // (16 * GRP), grp_body, 0)
        pltpu.make_async_copy(
            out_v.at[pl.ds(slot, 1)],
            out_hbm.at[pl.ds(fd, 1), pl.ds(c * OCH, OCH)],
            osems.at[slot]).start()
      return f

    lax.fori_loop(0, RPW, row_step, row0 // D)
    for slot in range(2):
      pltpu.make_async_copy(
          out_v.at[pl.ds(slot, 1)],
          out_hbm.at[pl.ds(0, 1), pl.ds(0, OCH)], osems.at[slot]).wait()

  return k(tbl_t, idx)


def _tc_dense_t(embT, W1, b1c, W2, b2c, W3, b3c, W4, b4c, S):
  bB = 1024
  grid = (B // bB,)
  dn0 = (((0,), (0,)), ((), ()))

  def body(x_ref, w1_ref, b1_ref, w2_ref, b2_ref, w3_ref, b3_ref, w4_ref,
           b4_ref, s_ref, o_ref):
    x = x_ref[...]
    sum_e = lax.dot_general(s_ref[...], x, dn0,
                            preferred_element_type=jnp.float32)
    sq_e = lax.dot_general(s_ref[...], x * x, dn0,
                           preferred_element_type=jnp.float32)
    fm = 0.5 * jnp.sum(sum_e * sum_e - sq_e, axis=0, keepdims=True)
    h = jnp.maximum(
        lax.dot_general(w1_ref[...], x, dn0,
                        preferred_element_type=jnp.float32) + b1_ref[...], 0.0)
    h = jnp.maximum(
        lax.dot_general(w2_ref[...], h, dn0,
                        preferred_element_type=jnp.float32) + b2_ref[...], 0.0)
    h = jnp.maximum(
        lax.dot_general(w3_ref[...], h, dn0,
                        preferred_element_type=jnp.float32) + b3_ref[...], 0.0)
    deep = lax.dot_general(w4_ref[...], h, dn0,
                           preferred_element_type=jnp.float32)
    o_ref[...] = fm + deep + b4_ref[...]

  full = lambda shape: pl.BlockSpec(shape, lambda i: (0,) * len(shape))
  return pl.pallas_call(
      body,
      grid=grid,
      in_specs=[
          pl.BlockSpec((MLP_IN, bB), lambda i: (0, i)),
          full((MLP_IN, 256)),
          full((256, 1)),
          full((256, 128)),
          full((128, 1)),
          full((128, 64)),
          full((64, 1)),
          full((64, 1)),
          full((1, 1)),
          full((MLP_IN, D)),
      ],
      out_specs=pl.BlockSpec((1, bB), lambda i: (0, i)),
      out_shape=jax.ShapeDtypeStruct((1, B), jnp.float32),
  )(embT, W1, b1c, W2, b2c, W3, b3c, W4, b4c, S)


@jax.jit
def kernel(sparse_indices_list, tables, W1, b1, W2, b2, W3, b3, W4, b4):
  tbl_t = tables.transpose(0, 2, 1).reshape(MLP_IN, V)
  idx = sparse_indices_list.astype(jnp.int32)

  embT = _sc_gather_t(tbl_t, idx)

  S = jnp.tile(jnp.eye(D, dtype=jnp.float32), (F, 1))
  logits = _tc_dense_t(embT, W1, b1.reshape(256, 1), W2, b2.reshape(128, 1),
                       W3, b3.reshape(64, 1), W4, b4.reshape(1, 1), S)
  return logits.reshape(B)


# speedup vs baseline: 1.0378x; 1.0070x over previous
import functools

import jax
import jax.numpy as jnp
from jax import lax
from jax.experimental import pallas as pl
from jax.experimental.pallas import tpu as pltpu
from jax.experimental.pallas import tpu_sc as plsc

F = 26
B = 16384
V = 100000
D = 64
MLP_IN = F * D

NC = 2
NS = 16
NW = NC * NS
RPW = MLP_IN // NW
OCH = 4096
NOC = B // OCH
GRP = 8


def _sc_gather_t(tbl_t, idx):
  mesh = plsc.VectorSubcoreMesh(
      core_axis_name="c", subcore_axis_name="s", num_cores=NC, num_subcores=NS)

  @functools.partial(
      pl.kernel,
      mesh=mesh,
      out_type=jax.ShapeDtypeStruct((MLP_IN, B), jnp.float32),
      scratch_types=[
          pltpu.VMEM((1, V), jnp.float32),
          pltpu.VMEM((1, B), jnp.int32),
          pltpu.VMEM((2, OCH), jnp.float32),
          pltpu.SemaphoreType.DMA,
          pltpu.SemaphoreType.DMA,
          pltpu.SemaphoreType.DMA((2,)),
      ],
      compiler_params=pltpu.CompilerParams(
          use_tc_tiling_on_sc=True, needs_layout_passes=False),
  )
  def k(tbl_hbm, idx_hbm, out_hbm, row_v, idx_v, out_v, rsem, isem, osems):
    wid = lax.axis_index("s") * NC + lax.axis_index("c")
    row0 = wid * RPW

    def load_idx(f):
      pltpu.make_async_copy(idx_hbm.at[pl.ds(f, 1)], idx_v, isem).start()
      pltpu.make_async_copy(idx_hbm.at[pl.ds(f, 1)], idx_v, isem).wait()

    load_idx(row0 // D)

    def row_step(r, f_loaded):
      fd = row0 + r
      f = fd // D
      pltpu.make_async_copy(tbl_hbm.at[pl.ds(fd, 1)], row_v, rsem).start()

      @pl.when(f != f_loaded)
      def _():
        load_idx(f)

      pltpu.make_async_copy(tbl_hbm.at[pl.ds(fd, 1)], row_v, rsem).wait()

      for c in range(NOC):
        slot = c % 2
        @pl.when((fd > row0) | (c >= 2))
        def _():
          pltpu.make_async_copy(
              out_v.at[pl.ds(slot, 1)],
              out_hbm.at[pl.ds(0, 1), pl.ds(0, OCH)], osems.at[slot]).wait()

        def grp_body(j, carry):
          ids_l = [idx_v[0, pl.ds(c * OCH + (j * GRP + u) * 16, 16)]
                   for u in range(GRP)]
          vals_l = [plsc.load_gather(row_v.at[0], [ids]) for ids in ids_l]
          for u in range(GRP):
            out_v[slot, pl.ds((j * GRP + u) * 16, 16)] = vals_l[u]
          return carry

        lax.fori_loop(0, OCH // (16 * GRP), grp_body, 0)
        pltpu.make_async_copy(
            out_v.at[pl.ds(slot, 1)],
            out_hbm.at[pl.ds(fd, 1), pl.ds(c * OCH, OCH)],
            osems.at[slot]).start()
      return f

    lax.fori_loop(0, RPW, row_step, row0 // D)
    for slot in range(2):
      pltpu.make_async_copy(
          out_v.at[pl.ds(slot, 1)],
          out_hbm.at[pl.ds(0, 1), pl.ds(0, OCH)], osems.at[slot]).wait()

  return k(tbl_t, idx)


def _tc_dense_t(embT, W1, b1c, W2, b2c, W3, b3c, W4, b4c, S):
  bB = 2048
  grid = (B // bB,)
  dn0 = (((0,), (0,)), ((), ()))

  def body(x_ref, w1_ref, b1_ref, w2_ref, b2_ref, w3_ref, b3_ref, w4_ref,
           b4_ref, s_ref, o_ref):
    x = x_ref[...]
    sum_e = lax.dot_general(s_ref[...], x, dn0,
                            preferred_element_type=jnp.float32)
    sq_e = lax.dot_general(s_ref[...], x * x, dn0,
                           preferred_element_type=jnp.float32)
    fm = 0.5 * jnp.sum(sum_e * sum_e - sq_e, axis=0, keepdims=True)
    h = jnp.maximum(
        lax.dot_general(w1_ref[...], x, dn0,
                        preferred_element_type=jnp.float32) + b1_ref[...], 0.0)
    h = jnp.maximum(
        lax.dot_general(w2_ref[...], h, dn0,
                        preferred_element_type=jnp.float32) + b2_ref[...], 0.0)
    h = jnp.maximum(
        lax.dot_general(w3_ref[...], h, dn0,
                        preferred_element_type=jnp.float32) + b3_ref[...], 0.0)
    deep = lax.dot_general(w4_ref[...], h, dn0,
                           preferred_element_type=jnp.float32)
    o_ref[...] = fm + deep + b4_ref[...]

  full = lambda shape: pl.BlockSpec(shape, lambda i: (0,) * len(shape))
  return pl.pallas_call(
      body,
      grid=grid,
      in_specs=[
          pl.BlockSpec((MLP_IN, bB), lambda i: (0, i)),
          full((MLP_IN, 256)),
          full((256, 1)),
          full((256, 128)),
          full((128, 1)),
          full((128, 64)),
          full((64, 1)),
          full((64, 1)),
          full((1, 1)),
          full((MLP_IN, D)),
      ],
      out_specs=pl.BlockSpec((1, bB), lambda i: (0, i)),
      out_shape=jax.ShapeDtypeStruct((1, B), jnp.float32),
  )(embT, W1, b1c, W2, b2c, W3, b3c, W4, b4c, S)


@jax.jit
def kernel(sparse_indices_list, tables, W1, b1, W2, b2, W3, b3, W4, b4):
  tbl_t = tables.transpose(0, 2, 1).reshape(MLP_IN, V)
  idx = sparse_indices_list.astype(jnp.int32)

  embT = _sc_gather_t(tbl_t, idx)

  S = jnp.tile(jnp.eye(D, dtype=jnp.float32), (F, 1))
  logits = _tc_dense_t(embT, W1, b1.reshape(256, 1), W2, b2.reshape(128, 1),
                       W3, b3.reshape(64, 1), W4, b4.reshape(1, 1), S)
  return logits.reshape(B)
